# Initial kernel scaffold; baseline (speedup 1.0000x reference)
import jax
import jax.numpy as jnp
from jax.experimental import pallas as pl


def kernel(x, table, W):
    raise NotImplementedError("write your pallas kernel here")


# speedup vs baseline: 1.0889x; 1.0889x over previous
import functools

import jax
import jax.numpy as jnp
from jax import lax
from jax.experimental import pallas as pl
from jax.experimental.pallas import tpu as pltpu
from jax.experimental.pallas import tpu_sc as plsc

VOCAB = 1000000
EMB = 64
OUT = 64
BATCH = 16384
HIST = 50
BTOT = BATCH * HIST


_NC, _NS = 2, 16
NW = _NC * _NS
B_PER_W = BTOT // NW
CHUNK = 512
N_CHUNKS = B_PER_W // CHUNK


def _sc_gather_body(table_hbm, idx_hbm, out_hbm, idx_v, rows_v, gsem, wsem):
    wid = lax.axis_index("s") * _NC + lax.axis_index("c")
    base = wid * B_PER_W
    pltpu.sync_copy(idx_hbm.at[pl.ds(base, B_PER_W)], idx_v)

    def chunk(c, carry):
        off = pl.multiple_of(c * CHUNK, CHUNK)
        pltpu.async_copy(
            table_hbm.at[idx_v.at[pl.ds(off, CHUNK)]], rows_v, gsem
        ).wait()
        pltpu.async_copy(rows_v, out_hbm.at[pl.ds(base + off, CHUNK)], wsem).wait()
        return carry

    lax.fori_loop(0, N_CHUNKS, chunk, 0)


def _sc_gather(table, idx):
    mesh = plsc.VectorSubcoreMesh(core_axis_name="c", subcore_axis_name="s")
    f = pl.kernel(
        _sc_gather_body,
        mesh=mesh,
        out_type=jax.ShapeDtypeStruct((BTOT, EMB), jnp.float32),
        scratch_types=[
            pltpu.VMEM((B_PER_W,), jnp.int32),
            pltpu.VMEM((CHUNK, EMB), jnp.float32),
            pltpu.SemaphoreType.DMA,
            pltpu.SemaphoreType.DMA,
        ],
        compiler_params=pltpu.CompilerParams(use_tc_tiling_on_sc=False),
    )
    return f(table, idx)


BLOCK_R = 2048


def _mm_body(emb_ref, wt_ref, out_ref):
    out_ref[...] = jnp.maximum(
        jnp.dot(emb_ref[...], wt_ref[...], preferred_element_type=jnp.float32),
        0.0,
    )


def _tc_matmul_relu(emb, wt):
    return pl.pallas_call(
        _mm_body,
        grid=(BTOT // BLOCK_R,),
        in_specs=[
            pl.BlockSpec((BLOCK_R, EMB), lambda i: (i, 0)),
            pl.BlockSpec((EMB, OUT), lambda i: (0, 0)),
        ],
        out_specs=pl.BlockSpec((BLOCK_R, OUT), lambda i: (i, 0)),
        out_shape=jax.ShapeDtypeStruct((BTOT, OUT), jnp.float32),
    )(emb, wt)


def kernel(x, table, W):
    idx = x.reshape(-1)
    emb = _sc_gather(table, idx)
    out = _tc_matmul_relu(emb, W.T)
    return out.reshape(BATCH, HIST, OUT)
